# baseline (speedup 1.0000x reference)
import jax
import jax.numpy as jnp
from jax import lax
from jax.experimental import pallas as pl
from jax.experimental.pallas import tpu as pltpu
from jax.experimental.pallas import tpu_sc as plsc

DEPTH = 1000
N_ROWS = 16384

_NC = 2
_NS = 16
_NW = _NC * _NS
_ROWS_PER_W = N_ROWS // _NW
_CHUNK = 32
_NCHUNK = _ROWS_PER_W // _CHUNK
_GROUPS = _CHUNK // 16


def _onehot_sc(idx_hbm, out_hbm, idx_v, buf0, buf1, sem0, sem1):
    wid = lax.axis_index("s") * _NC + lax.axis_index("c")
    base_row = wid * _ROWS_PER_W

    pltpu.sync_copy(idx_hbm.at[pl.ds(base_row, _ROWS_PER_W)], idx_v)

    zeros16 = jnp.zeros((16,), jnp.float32)

    def zero_body(i, _):
        off = pl.multiple_of(i * 16, 16)
        buf0[pl.ds(off, 16)] = zeros16
        buf1[pl.ds(off, 16)] = zeros16
        return 0

    lax.fori_loop(0, (_CHUNK * DEPTH) // 16, zero_body, 0)

    lane = jnp.arange(16, dtype=jnp.int32)
    ones16 = jnp.ones((16,), jnp.float32)
    bufs = (buf0, buf1)
    sems = (sem0, sem1)
    copies = [None, None]

    for c in range(_NCHUNK):
        b = c & 1
        buf = bufs[b]
        if c >= 2:
            copies[b].wait()
            for g in range(_GROUPS):
                vals = idx_v[pl.ds((c - 2) * _CHUNK + g * 16, 16)]
                pos = (lane + g * 16) * DEPTH + vals
                plsc.store_scatter(buf, [pos], zeros16)
        for g in range(_GROUPS):
            vals = idx_v[pl.ds(c * _CHUNK + g * 16, 16)]
            pos = (lane + g * 16) * DEPTH + vals
            plsc.store_scatter(buf, [pos], ones16)
        dst = out_hbm.at[pl.ds((base_row + c * _CHUNK) * DEPTH, _CHUNK * DEPTH)]
        cp = pltpu.make_async_copy(buf, dst, sems[b])
        cp.start()
        copies[b] = cp

    copies[0].wait()
    copies[1].wait()


@jax.jit
def _onehot(idx_flat):
    mesh = plsc.VectorSubcoreMesh(core_axis_name="c", subcore_axis_name="s")
    out = pl.kernel(
        _onehot_sc,
        out_type=jax.ShapeDtypeStruct((N_ROWS * DEPTH,), jnp.float32),
        mesh=mesh,
        scratch_types=[
            pltpu.VMEM((_ROWS_PER_W,), jnp.int32),
            pltpu.VMEM((_CHUNK * DEPTH,), jnp.float32),
            pltpu.VMEM((_CHUNK * DEPTH,), jnp.float32),
            pltpu.SemaphoreType.DMA,
            pltpu.SemaphoreType.DMA,
        ],
        compiler_params=pltpu.CompilerParams(needs_layout_passes=False),
    )(idx_flat)
    return out.reshape(N_ROWS, DEPTH)


def kernel(inputs):
    idx_flat = inputs.astype(jnp.int32).reshape(-1)
    return _onehot(idx_flat)


# speedup vs baseline: 1.4404x; 1.4404x over previous
import jax
import jax.numpy as jnp
from jax import lax
from jax.experimental import pallas as pl
from jax.experimental.pallas import tpu as pltpu
from jax.experimental.pallas import tpu_sc as plsc

DEPTH = 1000
N_ROWS = 16384

_NC = 2
_NS = 16
_NW = _NC * _NS
_ROWS_PER_W = N_ROWS // _NW
_CHUNK = 32
_NCHUNK = _ROWS_PER_W // _CHUNK
_GROUPS = _CHUNK // 16


def _onehot_sc(idx_hbm, zeros_hbm, out_hbm, idx_v, buf0, buf1, sem0, sem1):
    wid = lax.axis_index("s") * _NC + lax.axis_index("c")
    base_row = wid * _ROWS_PER_W

    cp0 = pltpu.make_async_copy(zeros_hbm, buf0, sem0)
    cp1 = pltpu.make_async_copy(zeros_hbm, buf1, sem1)
    cp0.start()
    cp1.start()
    pltpu.sync_copy(idx_hbm.at[pl.ds(base_row, _ROWS_PER_W)], idx_v)
    cp0.wait()
    cp1.wait()

    lane = jnp.arange(16, dtype=jnp.int32)
    ones16 = jnp.ones((16,), jnp.float32)
    zeros16 = jnp.zeros((16,), jnp.float32)
    bufs = (buf0, buf1)
    sems = (sem0, sem1)
    copies = [None, None]

    for c in range(_NCHUNK):
        b = c & 1
        buf = bufs[b]
        if c >= 2:
            copies[b].wait()
            for g in range(_GROUPS):
                vals = idx_v[pl.ds((c - 2) * _CHUNK + g * 16, 16)]
                plsc.store_scatter(buf, [lane + g * 16, vals], zeros16)
        for g in range(_GROUPS):
            vals = idx_v[pl.ds(c * _CHUNK + g * 16, 16)]
            plsc.store_scatter(buf, [lane + g * 16, vals], ones16)
        dst = out_hbm.at[pl.ds(base_row + c * _CHUNK, _CHUNK)]
        cp = pltpu.make_async_copy(buf, dst, sems[b])
        cp.start()
        copies[b] = cp

    copies[0].wait()
    copies[1].wait()


@jax.jit
def _onehot(idx_flat):
    mesh = plsc.VectorSubcoreMesh(core_axis_name="c", subcore_axis_name="s")
    zeros = jnp.zeros((_CHUNK, DEPTH), jnp.float32)
    return pl.kernel(
        _onehot_sc,
        out_type=jax.ShapeDtypeStruct((N_ROWS, DEPTH), jnp.float32),
        mesh=mesh,
        scratch_types=[
            pltpu.VMEM((_ROWS_PER_W,), jnp.int32),
            pltpu.VMEM((_CHUNK, DEPTH), jnp.float32),
            pltpu.VMEM((_CHUNK, DEPTH), jnp.float32),
            pltpu.SemaphoreType.DMA,
            pltpu.SemaphoreType.DMA,
        ],
        compiler_params=pltpu.CompilerParams(needs_layout_passes=False),
    )(idx_flat, zeros)


def kernel(inputs):
    idx_flat = inputs.astype(jnp.int32).reshape(-1)
    return _onehot(idx_flat)


# speedup vs baseline: 1.4413x; 1.0006x over previous
import jax
import jax.numpy as jnp
from jax import lax
from jax.experimental import pallas as pl
from jax.experimental.pallas import tpu as pltpu
from jax.experimental.pallas import tpu_sc as plsc

DEPTH = 1000
N_ROWS = 16384

_NC = 2
_NS = 16
_NW = _NC * _NS
_ROWS_PER_W = N_ROWS // _NW
_CHUNK = 32
_NCHUNK = _ROWS_PER_W // _CHUNK
_GROUPS = _CHUNK // 16


def _onehot_sc(idx_hbm, zeros_hbm, out_hbm, idx_v, buf0, buf1, sem0, sem1):
    wid = lax.axis_index("s") * _NC + lax.axis_index("c")
    base_row = wid * _ROWS_PER_W

    cp0 = pltpu.make_async_copy(zeros_hbm, buf0, sem0)
    cp1 = pltpu.make_async_copy(zeros_hbm, buf1, sem1)
    cp0.start()
    cp1.start()
    pltpu.sync_copy(idx_hbm.at[pl.ds(base_row, _ROWS_PER_W)], idx_v)
    cp0.wait()
    cp1.wait()

    lane = jnp.arange(16, dtype=jnp.int32)
    ones16 = jnp.ones((16,), jnp.float32)
    zeros16 = jnp.zeros((16,), jnp.float32)
    bufs = (buf0, buf1)
    sems = (sem0, sem1)
    copies = [None, None]

    for c in range(_NCHUNK):
        b = c & 1
        buf = bufs[b]
        if c >= 2:
            copies[b].wait()
            for g in range(_GROUPS):
                vals = idx_v[pl.ds((c - 2) * _CHUNK + g * 16, 16)]
                plsc.store_scatter(buf, [lane + g * 16, vals], zeros16)
        for g in range(_GROUPS):
            vals = idx_v[pl.ds(c * _CHUNK + g * 16, 16)]
            plsc.store_scatter(buf, [lane + g * 16, vals], ones16)
        dst = out_hbm.at[pl.ds(base_row + c * _CHUNK, _CHUNK)]
        cp = pltpu.make_async_copy(buf, dst, sems[b])
        cp.start()
        copies[b] = cp

    copies[0].wait()
    copies[1].wait()


@jax.jit
def _onehot(idx_flat):
    mesh = plsc.VectorSubcoreMesh(core_axis_name="c", subcore_axis_name="s")
    zeros = jnp.zeros((_CHUNK, DEPTH), jnp.float32)
    return pl.kernel(
        _onehot_sc,
        out_type=jax.ShapeDtypeStruct((N_ROWS, DEPTH), jnp.float32),
        mesh=mesh,
        scratch_types=[
            pltpu.VMEM((_ROWS_PER_W,), jnp.int32),
            pltpu.VMEM((_CHUNK, DEPTH), jnp.float32),
            pltpu.VMEM((_CHUNK, DEPTH), jnp.float32),
            pltpu.SemaphoreType.DMA,
            pltpu.SemaphoreType.DMA,
        ],
        compiler_params=pltpu.CompilerParams(
            needs_layout_passes=False, use_tc_tiling_on_sc=True
        ),
    )(idx_flat, zeros)


def kernel(inputs):
    idx_flat = inputs.astype(jnp.int32).reshape(-1)
    return _onehot(idx_flat)


# speedup vs baseline: 3.0261x; 2.0996x over previous
import jax
import jax.numpy as jnp
from jax import lax
from jax.experimental import pallas as pl
from jax.experimental.pallas import tpu as pltpu
from jax.experimental.pallas import tpu_sc as plsc

DEPTH = 1000
N_ROWS = 16384

_NC = 2
_NS = 16
_NW = _NC * _NS
_COLS_PER_W = N_ROWS // _NW
_SUB = 128
_NSUB = _COLS_PER_W // _SUB
_GROUPS = _SUB // 16


def _onehot_sc(idx_hbm, zeros_hbm, outT_hbm, idx_v, buf, sem, semz):
    wid = lax.axis_index("s") * _NC + lax.axis_index("c")
    col0 = wid * _COLS_PER_W

    cpz = pltpu.make_async_copy(zeros_hbm, buf, semz)
    cpz.start()
    pltpu.sync_copy(idx_hbm.at[pl.ds(col0, _COLS_PER_W)], idx_v)
    cpz.wait()

    lane = jnp.arange(16, dtype=jnp.int32)
    ones16 = jnp.ones((16,), jnp.float32)
    zeros16 = jnp.zeros((16,), jnp.float32)
    prev = None

    for c in range(_NSUB):
        if c > 0:
            prev.wait()
            for g in range(_GROUPS):
                vals = idx_v[pl.ds((c - 1) * _SUB + g * 16, 16)]
                plsc.store_scatter(buf, [vals, lane + g * 16], zeros16)
        for g in range(_GROUPS):
            vals = idx_v[pl.ds(c * _SUB + g * 16, 16)]
            plsc.store_scatter(buf, [vals, lane + g * 16], ones16)
        dst = outT_hbm.at[pl.ds(0, DEPTH), pl.ds(col0 + c * _SUB, _SUB)]
        prev = pltpu.make_async_copy(buf, dst, sem)
        prev.start()

    prev.wait()


@jax.jit
def _onehot(idx_flat):
    mesh = plsc.VectorSubcoreMesh(core_axis_name="c", subcore_axis_name="s")
    zeros = jnp.zeros((DEPTH, _SUB), jnp.float32)
    out_t = pl.kernel(
        _onehot_sc,
        out_type=jax.ShapeDtypeStruct((DEPTH, N_ROWS), jnp.float32),
        mesh=mesh,
        scratch_types=[
            pltpu.VMEM((_COLS_PER_W,), jnp.int32),
            pltpu.VMEM((DEPTH, _SUB), jnp.float32),
            pltpu.SemaphoreType.DMA,
            pltpu.SemaphoreType.DMA,
        ],
        compiler_params=pltpu.CompilerParams(
            needs_layout_passes=False, use_tc_tiling_on_sc=True
        ),
    )(idx_flat, zeros)
    return out_t.T


def kernel(inputs):
    idx_flat = inputs.astype(jnp.int32).reshape(-1)
    return _onehot(idx_flat)
